# baseline (speedup 1.0000x reference)
import math

import jax
import jax.numpy as jnp
from jax import lax
from jax.experimental import pallas as pl
from jax.experimental.pallas import tpu as pltpu
from jax.experimental.pallas import tpu_sc as plsc

_TWO_PI = 2.0 * math.pi
_INV_TWO_PI = 1.0 / _TWO_PI
_RESOLUTION = 1024

_L = 16
_NW = 32
_CHUNK = 16384
_NBUF = 2


_MAGIC = float(3 * 2**22)
_SCALE = float(_RESOLUTION) * _INV_TWO_PI


def _compute_chunk(x_v, out_v, table_v, b):
    lane = lax.iota(jnp.int32, _L)

    @plsc.parallel_loop(0, _CHUNK // _L, unroll=8)
    def _(i):
        xv = x_v[b, pl.ds(i * _L, _L)]
        u = xv * _SCALE
        v = (u - 0.5) + _MAGIC
        idx = plsc.bitcast(v, jnp.int32) & (_RESOLUTION - 1)
        idx2 = (idx << 4) | lane
        out_v[b, pl.ds(i * _L, _L)] = plsc.load_gather(table_v, [idx2])


def _trig_body(x_hbm, table_hbm, out_hbm, x_v, out_v, table_v, *sems):
    in_sems, out_sems = sems[:_NBUF], sems[_NBUF:]
    n_per_w = x_hbm.shape[0] // _NW
    n_chunks = n_per_w // _CHUNK
    wid = lax.axis_index("s") * 2 + lax.axis_index("c")
    base = wid * n_per_w

    pltpu.sync_copy(table_hbm, table_v)

    h_in = [None] * n_chunks
    h_out = [None] * n_chunks
    for c in range(_NBUF):
        h_in[c] = pltpu.async_copy(
            x_hbm.at[pl.ds(base + c * _CHUNK, _CHUNK)], x_v.at[c], in_sems[c])
    for c in range(n_chunks):
        b = c % _NBUF
        h_in[c].wait()
        if c >= _NBUF:
            h_out[c - _NBUF].wait()
        _compute_chunk(x_v, out_v, table_v, b)
        h_out[c] = pltpu.async_copy(
            out_v.at[b], out_hbm.at[pl.ds(base + c * _CHUNK, _CHUNK)],
            out_sems[b])
        if c + _NBUF < n_chunks:
            h_in[c + _NBUF] = pltpu.async_copy(
                x_hbm.at[pl.ds(base + (c + _NBUF) * _CHUNK, _CHUNK)],
                x_v.at[b], in_sems[b])
    for c in range(max(0, n_chunks - _NBUF), n_chunks):
        h_out[c].wait()


def kernel(x, sin_lookup):
    n = x.size
    mesh = plsc.VectorSubcoreMesh(core_axis_name="c", subcore_axis_name="s")
    flat = pl.kernel(
        _trig_body,
        mesh=mesh,
        out_type=jax.ShapeDtypeStruct((n,), jnp.float32),
        scratch_types=[
            pltpu.VMEM((_NBUF, _CHUNK), jnp.float32),
            pltpu.VMEM((_NBUF, _CHUNK), jnp.float32),
            pltpu.VMEM((_RESOLUTION * _L,), jnp.float32),
        ] + [pltpu.SemaphoreType.DMA] * (2 * _NBUF),
        compiler_params=pltpu.CompilerParams(needs_layout_passes=False),
    )(x.reshape(n),
      jnp.tile(sin_lookup.astype(jnp.float32)[:, None], (1, _L)).reshape(-1))
    return flat.reshape(x.shape)


# speedup vs baseline: 1.0976x; 1.0976x over previous
import math

import jax
import jax.numpy as jnp
from jax import lax
from jax.experimental import pallas as pl
from jax.experimental.pallas import tpu as pltpu
from jax.experimental.pallas import tpu_sc as plsc

_TWO_PI = 2.0 * math.pi
_INV_TWO_PI = 1.0 / _TWO_PI
_RESOLUTION = 1024

_L = 16
_NW = 32
_CHUNK = 8192
_NBUF = 3


_MAGIC = float(3 * 2**22)
_SCALE = float(_RESOLUTION) * _INV_TWO_PI


def _compute_chunk(x_v, out_v, table_v):
    @plsc.parallel_loop(0, _CHUNK // _L, unroll=8)
    def _(i):
        xv = x_v[pl.ds(i * _L, _L)]
        u = xv * _SCALE
        v = (u - 0.5) + _MAGIC
        idx = plsc.bitcast(v, jnp.int32) & (_RESOLUTION - 1)
        out_v[pl.ds(i * _L, _L)] = plsc.load_gather(table_v, [idx])


def _trig_body(x_hbm, table_hbm, out_hbm, *scratch):
    x_bufs = scratch[:_NBUF]
    out_bufs = scratch[_NBUF:2 * _NBUF]
    table_v = scratch[2 * _NBUF]
    in_sems = scratch[2 * _NBUF + 1:2 * _NBUF + 1 + _NBUF]
    out_sems = scratch[2 * _NBUF + 1 + _NBUF:]
    n_per_w = x_hbm.shape[0] // _NW
    n_chunks = n_per_w // _CHUNK
    wid = lax.axis_index("s") * 2 + lax.axis_index("c")
    base = wid * n_per_w

    pltpu.sync_copy(table_hbm, table_v)

    h_in = [None] * n_chunks
    h_out = [None] * n_chunks
    for c in range(_NBUF):
        h_in[c] = pltpu.async_copy(
            x_hbm.at[pl.ds(base + c * _CHUNK, _CHUNK)], x_bufs[c], in_sems[c])
    for c in range(n_chunks):
        b = c % _NBUF
        h_in[c].wait()
        if c >= _NBUF:
            h_out[c - _NBUF].wait()
        _compute_chunk(x_bufs[b], out_bufs[b], table_v)
        h_out[c] = pltpu.async_copy(
            out_bufs[b], out_hbm.at[pl.ds(base + c * _CHUNK, _CHUNK)],
            out_sems[b])
        if c + _NBUF < n_chunks:
            h_in[c + _NBUF] = pltpu.async_copy(
                x_hbm.at[pl.ds(base + (c + _NBUF) * _CHUNK, _CHUNK)],
                x_bufs[b], in_sems[b])
    for c in range(max(0, n_chunks - _NBUF), n_chunks):
        h_out[c].wait()


def kernel(x, sin_lookup):
    n = x.size
    mesh = plsc.VectorSubcoreMesh(core_axis_name="c", subcore_axis_name="s")
    flat = pl.kernel(
        _trig_body,
        mesh=mesh,
        out_type=jax.ShapeDtypeStruct((n,), jnp.float32),
        scratch_types=(
            [pltpu.VMEM((_CHUNK,), jnp.float32)] * (2 * _NBUF)
            + [pltpu.VMEM((_RESOLUTION,), jnp.float32)]
            + [pltpu.SemaphoreType.DMA] * (2 * _NBUF)
        ),
        compiler_params=pltpu.CompilerParams(needs_layout_passes=False),
    )(x.reshape(n), sin_lookup.astype(jnp.float32))
    return flat.reshape(x.shape)
